# baseline (speedup 1.0000x reference)
import functools

import jax
import jax.numpy as jnp
from jax import lax
from jax.experimental import pallas as pl
from jax.experimental.pallas import tpu as pltpu
from jax.experimental.pallas import tpu_sc as plsc

N = 10000
D = 128
H = 128
E = 320000

NC = 2
NS = 16
NW = NC * NS

NP = 10240
EC = 128
CPW = 80
HC = CPW // 2
EPAD = NW * CPW * EC
DEAD = N + 16
RPT = NP // NS
ZCP = RPT // EC

_SC_MESH = dict(core_axis_name="c", subcore_axis_name="s")


@functools.partial(
    pl.kernel,
    mesh=plsc.VectorSubcoreMesh(**_SC_MESH),
    out_type=jax.ShapeDtypeStruct((NC, NP, 16), jnp.float32),
    scratch_types=[
        pltpu.VMEM((CPW, EC), jnp.int32),
        pltpu.VMEM((EC, 16), jnp.float32),
        pltpu.VMEM_SHARED((NP, 16), jnp.float32),
    ],
)
def _deg_kernel(dst_hbm, out_hbm, idx_d, buf, acc):
    c = lax.axis_index("c")
    s = lax.axis_index("s")
    w = s * NC + c

    def zrow(i, carry):
        buf[i, :] = jnp.zeros((16,), jnp.float32)
        return carry

    lax.fori_loop(0, EC, zrow, 0)
    base = s * RPT
    for t in range(ZCP):
        pltpu.sync_copy(buf, acc.at[pl.ds(base + t * EC, EC), :])

    def orow(i, carry):
        buf[i, :] = jnp.ones((16,), jnp.float32)
        return carry

    lax.fori_loop(0, EC, orow, 0)
    pltpu.sync_copy(dst_hbm.at[w], idx_d)
    plsc.subcore_barrier()

    def body(j, carry):
        pltpu.sync_copy(buf, acc.at[idx_d.at[j]], add=True)
        return carry

    lax.fori_loop(0, CPW, body, 0)
    plsc.subcore_barrier()
    pltpu.sync_copy(acc.at[pl.ds(base, RPT), :], out_hbm.at[c, pl.ds(base, RPT), :])


@functools.partial(
    pl.kernel,
    mesh=plsc.VectorSubcoreMesh(**_SC_MESH),
    out_type=jax.ShapeDtypeStruct((NC, NP, H), jnp.float32),
    scratch_types=[
        pltpu.VMEM((HC, EC), jnp.int32),
        pltpu.VMEM((HC, EC), jnp.int32),
        pltpu.VMEM((EC, H), jnp.float32),
        pltpu.VMEM((EC, H), jnp.float32),
        pltpu.VMEM_SHARED((NP, H), jnp.float32),
        pltpu.SemaphoreType.DMA,
        pltpu.SemaphoreType.DMA,
    ],
)
def _scatter_kernel(xs_hbm, src_a, src_b, dst_a, dst_b, out_hbm,
                    idx_s, idx_d, rows0, rows1, acc, sem0, sem1):
    c = lax.axis_index("c")
    s = lax.axis_index("s")
    w = s * NC + c

    def zrow(i, carry):
        for jj in range(H // 16):
            rows0[i, pl.ds(jj * 16, 16)] = jnp.zeros((16,), jnp.float32)
        return carry

    lax.fori_loop(0, EC, zrow, 0)
    base = s * RPT
    for t in range(ZCP):
        pltpu.sync_copy(rows0, acc.at[pl.ds(base + t * EC, EC), :])

    for p, (src_hbm, dst_hbm) in enumerate([(src_a, dst_a), (src_b, dst_b)]):
        pltpu.sync_copy(src_hbm.at[w], idx_s)
        pltpu.sync_copy(dst_hbm.at[w], idx_d)
        if p == 0:
            plsc.subcore_barrier()

        def body(j, carry):
            pltpu.sync_copy(xs_hbm.at[idx_s.at[j]], rows0)
            pltpu.sync_copy(rows0, acc.at[idx_d.at[j]], add=True)
            return carry

        lax.fori_loop(0, HC, body, 0)

    plsc.subcore_barrier()
    pltpu.sync_copy(acc.at[pl.ds(base, RPT), :], out_hbm.at[c, pl.ds(base, RPT), :])


BLK = 1000
GRID = N // BLK


def _tc1_body(d0, d1, x, w, xs_o, dinv_o):
    deg = d0[:, 0:1] + d1[:, 0:1] + 1.0
    dinv = lax.rsqrt(deg)
    xw = jnp.dot(x[...], w[...], preferred_element_type=jnp.float32)
    xs_o[...] = xw * dinv
    dinv_o[...] = jnp.broadcast_to(dinv, xw.shape)


def _tc2_body(p0, p1, xs, dinv, w, b, xs2_o):
    t = (p0[...] + p1[...] + xs[...]) * dinv[...]
    h = jnp.maximum(t + b[...], 0.0)
    xs2_o[...] = jnp.dot(h, w[...], preferred_element_type=jnp.float32) * dinv[...]


def _tc3_body(q0, q1, xs2, dinv, b, wf, bf, out_o):
    t = (q0[...] + q1[...] + xs2[...]) * dinv[...]
    h = jnp.maximum(t + b[...], 0.0)
    out_o[...] = jnp.dot(h, wf[...], preferred_element_type=jnp.float32) + bf[...]


_row_spec = lambda width: pl.BlockSpec((BLK, width), lambda i: (i, 0))
_full_spec = lambda r, cc: pl.BlockSpec((r, cc), lambda i: (0, 0))

_tc1 = pl.pallas_call(
    _tc1_body,
    grid=(GRID,),
    in_specs=[_row_spec(16), _row_spec(16), _row_spec(D), _full_spec(D, H)],
    out_specs=[_row_spec(H), _row_spec(H)],
    out_shape=[
        jax.ShapeDtypeStruct((N, H), jnp.float32),
        jax.ShapeDtypeStruct((N, H), jnp.float32),
    ],
)

_tc2 = pl.pallas_call(
    _tc2_body,
    grid=(GRID,),
    in_specs=[_row_spec(H), _row_spec(H), _row_spec(H), _row_spec(H),
              _full_spec(H, H), _full_spec(1, H)],
    out_specs=[_row_spec(H)],
    out_shape=[jax.ShapeDtypeStruct((N, H), jnp.float32)],
)

_tc3 = pl.pallas_call(
    _tc3_body,
    grid=(GRID,),
    in_specs=[_row_spec(H), _row_spec(H), _row_spec(H), _row_spec(H),
              _full_spec(1, H), _full_spec(H, 1), _full_spec(1, 1)],
    out_specs=[pl.BlockSpec((BLK, 1), lambda i: (i, 0))],
    out_shape=[jax.ShapeDtypeStruct((N, 1), jnp.float32)],
)


def kernel(x, edge_index, W1, b1, W2, b2, Wf, bf):
    pad_s = jnp.zeros((EPAD - E,), jnp.int32)
    pad_d = jnp.full((EPAD - E,), DEAD, jnp.int32)
    src4 = jnp.concatenate([edge_index[0], pad_s]).reshape(NW, 2, HC, EC)
    dst4 = jnp.concatenate([edge_index[1], pad_d]).reshape(NW, 2, HC, EC)
    src_a, src_b = src4[:, 0], src4[:, 1]
    dst_a, dst_b = dst4[:, 0], dst4[:, 1]
    dst3 = dst4.reshape(NW, CPW, EC)

    degp = _deg_kernel(dst3)
    xs1, dinv = _tc1(degp[0], degp[1], x, W1)
    p = _scatter_kernel(xs1, src_a, src_b, dst_a, dst_b)
    (xs2,) = _tc2(p[0], p[1], xs1, dinv, W2, b1.reshape(1, H))
    q = _scatter_kernel(xs2, src_a, src_b, dst_a, dst_b)
    (out,) = _tc3(q[0], q[1], xs2, dinv, b2.reshape(1, H),
                  Wf, bf.reshape(1, 1))
    return out.reshape(N)


# speedup vs baseline: 1.5515x; 1.5515x over previous
import functools

import jax
import jax.numpy as jnp
from jax import lax
from jax.experimental import pallas as pl
from jax.experimental.pallas import tpu as pltpu
from jax.experimental.pallas import tpu_sc as plsc

N = 10000
D = 128
H = 128
E = 320000

NC = 2
NS = 16
NW = NC * NS

NP = 10240
EC = 80
CPW = 126
HC = CPW // 2
EPAD = NW * CPW * EC
DEAD = N + 16
RPT = NP // NS
ZCP = RPT // EC

_SC_MESH = dict(core_axis_name="c", subcore_axis_name="s")


@functools.partial(
    pl.kernel,
    mesh=plsc.VectorSubcoreMesh(**_SC_MESH),
    out_type=jax.ShapeDtypeStruct((NC, NP, 16), jnp.float32),
    scratch_types=[
        pltpu.VMEM((CPW, EC), jnp.int32),
        pltpu.VMEM((EC, 16), jnp.float32),
        pltpu.VMEM_SHARED((NP, 16), jnp.float32),
    ],
)
def _deg_kernel(dst_hbm, out_hbm, idx_d, buf, acc):
    c = lax.axis_index("c")
    s = lax.axis_index("s")
    w = s * NC + c

    def zrow(i, carry):
        buf[i, :] = jnp.zeros((16,), jnp.float32)
        return carry

    lax.fori_loop(0, EC, zrow, 0)
    base = s * RPT
    for t in range(ZCP):
        pltpu.sync_copy(buf, acc.at[pl.ds(base + t * EC, EC), :])

    def orow(i, carry):
        buf[i, :] = jnp.ones((16,), jnp.float32)
        return carry

    lax.fori_loop(0, EC, orow, 0)
    pltpu.sync_copy(dst_hbm.at[w], idx_d)
    plsc.subcore_barrier()

    def body(j, carry):
        pltpu.sync_copy(buf, acc.at[idx_d.at[j]], add=True)
        return carry

    lax.fori_loop(0, CPW, body, 0)
    plsc.subcore_barrier()
    pltpu.sync_copy(acc.at[pl.ds(base, RPT), :], out_hbm.at[c, pl.ds(base, RPT), :])


@functools.partial(
    pl.kernel,
    mesh=plsc.VectorSubcoreMesh(**_SC_MESH),
    out_type=jax.ShapeDtypeStruct((NC, NP, H), jnp.float32),
    scratch_types=[
        pltpu.VMEM((HC, EC), jnp.int32),
        pltpu.VMEM((HC, EC), jnp.int32),
        pltpu.VMEM((EC, H), jnp.float32),
        pltpu.VMEM((EC, H), jnp.float32),
        pltpu.VMEM_SHARED((NP, H), jnp.float32),
        pltpu.SemaphoreType.DMA,
    ],
)
def _scatter_kernel(xs_hbm, src_a, src_b, dst_a, dst_b, out_hbm,
                    idx_s, idx_d, rows0, rows1, acc, sem0):
    c = lax.axis_index("c")
    s = lax.axis_index("s")
    w = s * NC + c

    def zrow(i, carry):
        for jj in range(H // 16):
            rows0[i, pl.ds(jj * 16, 16)] = jnp.zeros((16,), jnp.float32)
        return carry

    lax.fori_loop(0, EC, zrow, 0)
    base = s * RPT
    for t in range(ZCP):
        pltpu.sync_copy(rows0, acc.at[pl.ds(base + t * EC, EC), :])

    for p, (src_hbm, dst_hbm) in enumerate([(src_a, dst_a), (src_b, dst_b)]):
        pltpu.sync_copy(src_hbm.at[w], idx_s)
        pltpu.sync_copy(dst_hbm.at[w], idx_d)
        if p == 0:
            plsc.subcore_barrier()

        def body(i, carry):
            g = i * 2
            pltpu.sync_copy(xs_hbm.at[idx_s.at[g]], rows0)
            h = pltpu.async_copy(rows0, acc.at[idx_d.at[g]], sem0, add=True)
            pltpu.sync_copy(xs_hbm.at[idx_s.at[g + 1]], rows1)
            h.wait()
            pltpu.sync_copy(rows1, acc.at[idx_d.at[g + 1]], add=True)
            return carry

        lax.fori_loop(0, (HC - 1) // 2, body, 0)
        pltpu.sync_copy(xs_hbm.at[idx_s.at[HC - 1]], rows0)
        pltpu.sync_copy(rows0, acc.at[idx_d.at[HC - 1]], add=True)

    plsc.subcore_barrier()
    pltpu.sync_copy(acc.at[pl.ds(base, RPT), :], out_hbm.at[c, pl.ds(base, RPT), :])


BLK = 1000
GRID = N // BLK


def _tc1_body(d0, d1, x, w, xs_o, dinv_o):
    deg = d0[:, 0:1] + d1[:, 0:1] + 1.0
    dinv = lax.rsqrt(deg)
    xw = jnp.dot(x[...], w[...], preferred_element_type=jnp.float32)
    xs_o[...] = xw * dinv
    dinv_o[...] = jnp.broadcast_to(dinv, xw.shape)


def _tc2_body(p0, p1, xs, dinv, w, b, xs2_o):
    t = (p0[...] + p1[...] + xs[...]) * dinv[...]
    h = jnp.maximum(t + b[...], 0.0)
    xs2_o[...] = jnp.dot(h, w[...], preferred_element_type=jnp.float32) * dinv[...]


def _tc3_body(q0, q1, xs2, dinv, b, wf, bf, out_o):
    t = (q0[...] + q1[...] + xs2[...]) * dinv[...]
    h = jnp.maximum(t + b[...], 0.0)
    out_o[...] = jnp.dot(h, wf[...], preferred_element_type=jnp.float32) + bf[...]


_row_spec = lambda width: pl.BlockSpec((BLK, width), lambda i: (i, 0))
_full_spec = lambda r, cc: pl.BlockSpec((r, cc), lambda i: (0, 0))

_tc1 = pl.pallas_call(
    _tc1_body,
    grid=(GRID,),
    in_specs=[_row_spec(16), _row_spec(16), _row_spec(D), _full_spec(D, H)],
    out_specs=[_row_spec(H), _row_spec(H)],
    out_shape=[
        jax.ShapeDtypeStruct((N, H), jnp.float32),
        jax.ShapeDtypeStruct((N, H), jnp.float32),
    ],
)

_tc2 = pl.pallas_call(
    _tc2_body,
    grid=(GRID,),
    in_specs=[_row_spec(H), _row_spec(H), _row_spec(H), _row_spec(H),
              _full_spec(H, H), _full_spec(1, H)],
    out_specs=[_row_spec(H)],
    out_shape=[jax.ShapeDtypeStruct((N, H), jnp.float32)],
)

_tc3 = pl.pallas_call(
    _tc3_body,
    grid=(GRID,),
    in_specs=[_row_spec(H), _row_spec(H), _row_spec(H), _row_spec(H),
              _full_spec(1, H), _full_spec(H, 1), _full_spec(1, 1)],
    out_specs=[pl.BlockSpec((BLK, 1), lambda i: (i, 0))],
    out_shape=[jax.ShapeDtypeStruct((N, 1), jnp.float32)],
)


def kernel(x, edge_index, W1, b1, W2, b2, Wf, bf):
    pad_s = jnp.zeros((EPAD - E,), jnp.int32)
    pad_d = jnp.full((EPAD - E,), DEAD, jnp.int32)
    src4 = jnp.concatenate([edge_index[0], pad_s]).reshape(NW, 2, HC, EC)
    dst4 = jnp.concatenate([edge_index[1], pad_d]).reshape(NW, 2, HC, EC)
    src_a, src_b = src4[:, 0], src4[:, 1]
    dst_a, dst_b = dst4[:, 0], dst4[:, 1]
    dst3 = dst4.reshape(NW, CPW, EC)

    degp = _deg_kernel(dst3)
    xs1, dinv = _tc1(degp[0], degp[1], x, W1)
    p = _scatter_kernel(xs1, src_a, src_b, dst_a, dst_b)
    (xs2,) = _tc2(p[0], p[1], xs1, dinv, W2, b1.reshape(1, H))
    q = _scatter_kernel(xs2, src_a, src_b, dst_a, dst_b)
    (out,) = _tc3(q[0], q[1], xs2, dinv, b2.reshape(1, H),
                  Wf, bf.reshape(1, 1))
    return out.reshape(N)


# speedup vs baseline: 2.0334x; 1.3106x over previous
import functools

import jax
import jax.numpy as jnp
from jax import lax
from jax.experimental import pallas as pl
from jax.experimental.pallas import tpu as pltpu
from jax.experimental.pallas import tpu_sc as plsc

N = 10000
D = 128
H = 128
E = 320000

NC = 2
NS = 16
NW = NC * NS

NP = 10240
EC = 80
CPW = 125
EPW = CPW * EC
RPT = NP // NS
ZCP = RPT // EC

_SC_MESH = dict(core_axis_name="c", subcore_axis_name="s")


@functools.partial(
    pl.kernel,
    mesh=plsc.VectorSubcoreMesh(**_SC_MESH),
    out_type=jax.ShapeDtypeStruct((NC, NP, 16), jnp.float32),
    scratch_types=[
        pltpu.VMEM((CPW, EC), jnp.int32),
        pltpu.VMEM((EC, 16), jnp.float32),
        pltpu.VMEM_SHARED((NP, 16), jnp.float32),
    ],
)
def _deg_kernel(dst_hbm, out_hbm, idx_d, buf, acc):
    c = lax.axis_index("c")
    s = lax.axis_index("s")
    w = s * NC + c

    def zrow(i, carry):
        buf[i, :] = jnp.zeros((16,), jnp.float32)
        return carry

    lax.fori_loop(0, EC, zrow, 0)
    base = s * RPT
    for t in range(ZCP):
        pltpu.sync_copy(buf, acc.at[pl.ds(base + t * EC, EC), :])

    def orow(i, carry):
        buf[i, :] = jnp.ones((16,), jnp.float32)
        return carry

    lax.fori_loop(0, EC, orow, 0)
    pltpu.sync_copy(dst_hbm.at[w], idx_d)
    plsc.subcore_barrier()

    def body(j, carry):
        pltpu.sync_copy(buf, acc.at[idx_d.at[j]], add=True)
        return carry

    lax.fori_loop(0, CPW, body, 0)
    plsc.subcore_barrier()
    pltpu.sync_copy(acc.at[pl.ds(base, RPT), :], out_hbm.at[c, pl.ds(base, RPT), :])


@functools.partial(
    pl.kernel,
    mesh=plsc.VectorSubcoreMesh(**_SC_MESH),
    out_type=jax.ShapeDtypeStruct((NC, NP, H), jnp.float32),
    scratch_types=[
        pltpu.VMEM((CPW, EC), jnp.int32),
        pltpu.VMEM((CPW, EC), jnp.int32),
        pltpu.VMEM((EC, H), jnp.float32),
        pltpu.VMEM_SHARED((NP, H), jnp.float32),
    ],
)
def _scatter_kernel(xs_hbm, src_hbm, dst_hbm, out_hbm,
                    idx_s, idx_d, rows, acc):
    c = lax.axis_index("c")
    s = lax.axis_index("s")
    w = s * NC + c

    def zrow(i, carry):
        for jj in range(H // 16):
            rows[i, pl.ds(jj * 16, 16)] = jnp.zeros((16,), jnp.float32)
        return carry

    lax.fori_loop(0, EC, zrow, 0)
    base = s * RPT
    for t in range(ZCP):
        pltpu.sync_copy(rows, acc.at[pl.ds(base + t * EC, EC), :])

    pltpu.sync_copy(src_hbm.at[w], idx_s)
    pltpu.sync_copy(dst_hbm.at[w], idx_d)
    plsc.subcore_barrier()

    def body(j, carry):
        pltpu.sync_copy(xs_hbm.at[idx_s.at[j]], rows)
        pltpu.sync_copy(rows, acc.at[idx_d.at[j]], add=True)
        return carry

    lax.fori_loop(0, CPW, body, 0)

    plsc.subcore_barrier()
    pltpu.sync_copy(acc.at[pl.ds(base, RPT), :], out_hbm.at[c, pl.ds(base, RPT), :])


BLK = 1000
GRID = N // BLK


def _tc1_body(d0, d1, x, w, xs_o, dinv_o):
    deg = d0[:, 0:1] + d1[:, 0:1] + 1.0
    dinv = lax.rsqrt(deg)
    xw = jnp.dot(x[...], w[...], preferred_element_type=jnp.float32)
    xs_o[...] = xw * dinv
    dinv_o[...] = jnp.broadcast_to(dinv, xw.shape)


def _tc2_body(p0, p1, xs, dinv, w, b, xs2_o):
    t = (p0[...] + p1[...] + xs[...]) * dinv[...]
    h = jnp.maximum(t + b[...], 0.0)
    xs2_o[...] = jnp.dot(h, w[...], preferred_element_type=jnp.float32) * dinv[...]


def _tc3_body(q0, q1, xs2, dinv, b, wf, bf, out_o):
    t = (q0[...] + q1[...] + xs2[...]) * dinv[...]
    h = jnp.maximum(t + b[...], 0.0)
    out_o[...] = jnp.dot(h, wf[...], preferred_element_type=jnp.float32) + bf[...]


_row_spec = lambda width: pl.BlockSpec((BLK, width), lambda i: (i, 0))
_full_spec = lambda r, cc: pl.BlockSpec((r, cc), lambda i: (0, 0))

_tc1 = pl.pallas_call(
    _tc1_body,
    grid=(GRID,),
    in_specs=[_row_spec(16), _row_spec(16), _row_spec(D), _full_spec(D, H)],
    out_specs=[_row_spec(H), _row_spec(H)],
    out_shape=[
        jax.ShapeDtypeStruct((N, H), jnp.float32),
        jax.ShapeDtypeStruct((N, H), jnp.float32),
    ],
)

_tc2 = pl.pallas_call(
    _tc2_body,
    grid=(GRID,),
    in_specs=[_row_spec(H), _row_spec(H), _row_spec(H), _row_spec(H),
              _full_spec(H, H), _full_spec(1, H)],
    out_specs=[_row_spec(H)],
    out_shape=[jax.ShapeDtypeStruct((N, H), jnp.float32)],
)

_tc3 = pl.pallas_call(
    _tc3_body,
    grid=(GRID,),
    in_specs=[_row_spec(H), _row_spec(H), _row_spec(H), _row_spec(H),
              _full_spec(1, H), _full_spec(H, 1), _full_spec(1, 1)],
    out_specs=[pl.BlockSpec((BLK, 1), lambda i: (i, 0))],
    out_shape=[jax.ShapeDtypeStruct((N, 1), jnp.float32)],
)


def kernel(x, edge_index, W1, b1, W2, b2, Wf, bf):
    src3 = edge_index[0].reshape(NW, CPW, EC)
    dst3 = edge_index[1].reshape(NW, CPW, EC)

    degp = _deg_kernel(dst3)
    xs1, dinv = _tc1(degp[0], degp[1], x, W1)
    p = _scatter_kernel(xs1, src3, dst3)
    (xs2,) = _tc2(p[0], p[1], xs1, dinv, W2, b1.reshape(1, H))
    q = _scatter_kernel(xs2, src3, dst3)
    (out,) = _tc3(q[0], q[1], xs2, dinv, b2.reshape(1, H),
                  Wf, bf.reshape(1, 1))
    return out.reshape(N)
